# baseline (speedup 1.0000x reference)
import functools

import jax
import jax.numpy as jnp
from jax import lax
from jax.experimental import pallas as pl
from jax.experimental.pallas import tpu as pltpu
from jax.experimental.pallas import tpu_sc as plsc

_PH = 7
_PW = 7
_SCALE = 224.0
_BINS = _PH * _PW
_RPB = 16
_R = _BINS * _RPB

_NC = 2
_NS = 16
_NW = _NC * _NS
_LANES = 16

_CHUNK_BINS = 7
_CHUNK_ROWS = _CHUNK_BINS * _RPB
_NCHUNKS = _BINS // _CHUNK_BINS


def _coef_body(rois_ref, idx_ref, wgt_ref, *, size_h, size_w, n_hw):
    rois = rois_ref[...]
    r = lax.broadcasted_iota(jnp.int32, (1, _R), 1)
    bin_id = r // _RPB
    ph = (bin_id // _PW).astype(jnp.float32)
    pw = (bin_id % _PW).astype(jnp.float32)
    j = r % _RPB
    sy = (j // 8).astype(jnp.float32)
    sx = ((j // 4) % 2).astype(jnp.float32)
    cy = (j // 2) % 2
    cx = j % 2

    batch = rois[:, 0:1].astype(jnp.int32)
    start_w = rois[:, 1:2] * _SCALE
    start_h = rois[:, 2:3] * _SCALE
    end_w = rois[:, 3:4] * _SCALE
    end_h = rois[:, 4:5] * _SCALE
    roi_w = jnp.maximum(end_w - start_w, 1.0)
    roi_h = jnp.maximum(end_h - start_h, 1.0)
    bin_h = roi_h / _PH
    bin_w = roi_w / _PW
    y = start_h + (ph + (sy + 0.5) * 0.5) * bin_h
    x = start_w + (pw + (sx + 0.5) * 0.5) * bin_w

    def axis(coord, size, take_high):
        valid = (coord >= -1.0) & (coord <= float(size))
        c = jnp.maximum(coord, 0.0)
        low = jnp.minimum(jnp.floor(c).astype(jnp.int32), size - 1)
        high = jnp.minimum(low + 1, size - 1)
        frac = jnp.where(low >= size - 1, 0.0, c - low.astype(jnp.float32))
        sel = jnp.where(take_high == 1, high, low)
        wgt = jnp.where(take_high == 1, frac, 1.0 - frac)
        return sel, wgt, valid

    ysel, wy, vy = axis(y, size_h, cy)
    xsel, wx, vx = axis(x, size_w, cx)
    w = 0.25 * wy * wx * (vy & vx).astype(jnp.float32)
    idx_ref[...] = batch * n_hw + ysel * size_w + xsel
    wgt_ref[...] = w


def _coef(rois, size_h, size_w, n_hw):
    k = rois.shape[0]
    return pl.pallas_call(
        functools.partial(_coef_body, size_h=size_h, size_w=size_w, n_hw=n_hw),
        out_shape=[
            jax.ShapeDtypeStruct((k, _R), jnp.int32),
            jax.ShapeDtypeStruct((k, _R), jnp.float32),
        ],
    )(rois)


_XSEG = 112


def _make_sc_transpose(n, c, h, w):
    nseg = w // _XSEG
    units = n * h * nseg
    upt = units // _NW
    npairs = upt // 2
    mesh = plsc.VectorSubcoreMesh(
        core_axis_name="c", subcore_axis_name="s",
        num_cores=_NC, num_subcores=_NS)

    @functools.partial(
        pl.kernel,
        out_type=jax.ShapeDtypeStruct((n * h * w * c // 2,), jnp.int32),
        mesh=mesh,
        compiler_params=pltpu.CompilerParams(
            needs_layout_passes=False, use_tc_tiling_on_sc=False),
        scratch_types=[
            pltpu.VMEM((2, c, _XSEG), jnp.float32),
            pltpu.VMEM((2, _XSEG * c // 2), jnp.int32),
            pltpu.SemaphoreType.DMA,
            pltpu.SemaphoreType.DMA,
            pltpu.SemaphoreType.DMA,
            pltpu.SemaphoreType.DMA,
        ],
    )
    def tr_fn(in4, table, vin, vout, isem0, isem1, osem0, osem1):
        wid = lax.axis_index("s") * _NC + lax.axis_index("c")
        base = wid * upt
        isems = (isem0, isem1)
        osems = (osem0, osem1)

        def src_ref(u):
            uid = base + u
            nn = uid // (h * nseg)
            rem = lax.rem(uid, h * nseg)
            yy = rem // nseg
            xh = lax.rem(rem, nseg)
            return in4.at[nn, :, yy, pl.ds(xh * _XSEG, _XSEG)]

        def dst_ref(u):
            uid = base + u
            nn = uid // (h * nseg)
            rem = lax.rem(uid, h * nseg)
            yy = rem // nseg
            xh = lax.rem(rem, nseg)
            pix0 = (nn * h + yy) * w + xh * _XSEG
            return table.at[pl.ds(pix0 * (c // 2), _XSEG * (c // 2))]

        pltpu.async_copy(src_ref(0), vin.at[0], isems[0])
        pltpu.async_copy(src_ref(1), vin.at[1], isems[1])

        def pair_body(s, carry):
            for b in range(2):
                u = 2 * s + b
                pltpu.make_async_copy(src_ref(u), vin.at[b], isems[b]).wait()

                @pl.when(s >= 1)
                def _wait_prev_out(b=b, s=s):
                    pltpu.make_async_copy(
                        vout.at[b], dst_ref(2 * (s - 1) + b),
                        osems[b]).wait()

                @plsc.parallel_loop(0, _XSEG // _LANES)
                def pg_body(pg, b=b):
                    rowbase = (pg * _LANES
                               + lax.iota(jnp.int32, _LANES)) * (c // 2)
                    for ch in range(0, c, 2):
                        va = vin[b, ch, pl.ds(pg * _LANES, _LANES)]
                        vb = vin[b, ch + 1, pl.ds(pg * _LANES, _LANES)]
                        packed = plsc.pack(
                            va, vb, format=plsc.PackFormat.INTERLEAVED)
                        word = plsc.bitcast(packed, jnp.int32)
                        plsc.store_scatter(
                            vout.at[b], [rowbase + (ch // 2)], word)

                @pl.when(u + 2 < upt)
                def _issue_next(u=u, b=b):
                    pltpu.async_copy(src_ref(u + 2), vin.at[b], isems[b])

                pltpu.async_copy(vout.at[b], dst_ref(u), osems[b])
            return carry

        lax.fori_loop(0, npairs, pair_body, 0)
        for b in range(2):
            pltpu.make_async_copy(
                vout.at[b], dst_ref(upt - 2 + b), osems[b]).wait()

    return tr_fn


def _make_sc_kernel(k_rois, channels):
    rois_per_w = k_rois // _NW
    cpairs = channels // (2 * _LANES)
    mesh = plsc.VectorSubcoreMesh(
        core_axis_name="c", subcore_axis_name="s",
        num_cores=_NC, num_subcores=_NS)

    @functools.partial(
        pl.kernel,
        out_type=jax.ShapeDtypeStruct((k_rois, _BINS * channels), jnp.float32),
        mesh=mesh,
        compiler_params=pltpu.CompilerParams(
            needs_layout_passes=False, use_tc_tiling_on_sc=False),
        scratch_types=[
            pltpu.VMEM((2, _NCHUNKS, _CHUNK_ROWS), jnp.int32),
            pltpu.VMEM((2, _R), jnp.float32),
            pltpu.VMEM((3, _CHUNK_ROWS, channels // 2), jnp.int32),
            pltpu.VMEM((_BINS * channels,), jnp.float32),
            pltpu.SemaphoreType.DMA,
            pltpu.SemaphoreType.DMA,
            pltpu.SemaphoreType.DMA,
            pltpu.SemaphoreType.DMA,
            pltpu.SemaphoreType.DMA,
        ],
    )
    def sc_fn(table, idxs, wgts, out, idx_v, wgt_v, gbuf, obuf,
              isem, wsem, gsem0, gsem1, gsem2):
        wid = lax.axis_index("s") * _NC + lax.axis_index("c")
        k0 = wid * rois_per_w
        gsems = (gsem0, gsem1, gsem2)

        pltpu.async_copy(idxs.at[k0], idx_v.at[0], isem)
        pltpu.async_copy(wgts.at[k0], wgt_v.at[0], wsem)

        def roi_body(i, carry):
            k = k0 + i
            ib = lax.rem(i, 2)
            nib = 1 - ib
            pltpu.make_async_copy(idxs.at[k], idx_v.at[ib], isem).wait()
            pltpu.make_async_copy(wgts.at[k], wgt_v.at[ib], wsem).wait()

            @pl.when(i + 1 < rois_per_w)
            def _prefetch():
                pltpu.async_copy(idxs.at[k + 1], idx_v.at[nib], isem)
                pltpu.async_copy(wgts.at[k + 1], wgt_v.at[nib], wsem)

            descs = {}
            descs[0] = pltpu.async_copy(
                table.at[idx_v.at[ib, 0]], gbuf.at[0], gsems[0])
            descs[1] = pltpu.async_copy(
                table.at[idx_v.at[ib, 1]], gbuf.at[1], gsems[1])

            for c in range(_NCHUNKS):
                buf = c % 3
                if c + 2 < _NCHUNKS:
                    nb = (c + 2) % 3
                    descs[c + 2] = pltpu.async_copy(
                        table.at[idx_v.at[ib, c + 2]], gbuf.at[nb], gsems[nb])
                descs[c].wait()

                @plsc.parallel_loop(0, _CHUNK_BINS)
                def bin_body(bl, c=c, buf=buf, ib=ib):
                    b = c * _CHUNK_BINS + bl
                    base_row = bl * _RPB
                    base_r = b * _RPB
                    acc_a = [jnp.zeros((_LANES,), jnp.float32)
                             for _ in range(cpairs)]
                    acc_b = [jnp.zeros((_LANES,), jnp.float32)
                             for _ in range(cpairs)]
                    for jj in range(_RPB):
                        wj = plsc.load_gather(
                            wgt_v.at[ib],
                            [jnp.broadcast_to(base_r + jj, (_LANES,))])
                        row = base_row + jj
                        for ci in range(cpairs):
                            word = gbuf[buf, row, pl.ds(ci * _LANES, _LANES)]
                            ea, eb = plsc.unpack(
                                plsc.bitcast(word, jnp.bfloat16),
                                format=plsc.PackFormat.INTERLEAVED,
                                preferred_element_type=jnp.float32)
                            acc_a[ci] = acc_a[ci] + wj * ea
                            acc_b[ci] = acc_b[ci] + wj * eb
                    ev2 = 2 * lax.iota(jnp.int32, _LANES)
                    for ci in range(cpairs):
                        cbase = b * channels + ci * 2 * _LANES
                        plsc.store_scatter(obuf, [cbase + ev2], acc_a[ci])
                        plsc.store_scatter(obuf, [cbase + ev2 + 1], acc_b[ci])

            pltpu.sync_copy(obuf, out.at[k])
            return carry

        lax.fori_loop(0, rois_per_w, roi_body, 0)

    return sc_fn


@jax.jit
def kernel(input, rois):
    n, c, h, w = input.shape
    k = rois.shape[0]
    in_lin = jax.lax.optimization_barrier(input.reshape(-1))
    in4 = in_lin.reshape(n, c, h, w)
    table = _make_sc_transpose(n, c, h, w)(in4).reshape(n * h * w, c // 2)
    idx, wgt = _coef(rois, h, w, h * w)
    idx3 = jax.lax.optimization_barrier(
        idx.reshape(-1)).reshape(k, _NCHUNKS, _CHUNK_ROWS)
    wgt2 = jax.lax.optimization_barrier(wgt.reshape(-1)).reshape(k, _R)
    out = _make_sc_kernel(k, c)(table, idx3, wgt2).reshape(k, _BINS, c)
    return out.transpose(0, 2, 1).reshape(k, c, _PH, _PW)


# speedup vs baseline: 1.1265x; 1.1265x over previous
import functools

import jax
import jax.numpy as jnp
from jax import lax
from jax.experimental import pallas as pl
from jax.experimental.pallas import tpu as pltpu
from jax.experimental.pallas import tpu_sc as plsc

_PH = 7
_PW = 7
_SCALE = 224.0
_BINS = _PH * _PW
_RPB = 16
_R = _BINS * _RPB

_NC = 2
_NS = 16
_NW = _NC * _NS
_LANES = 16

_CHUNK_BINS = 7
_CHUNK_ROWS = _CHUNK_BINS * _RPB
_NCHUNKS = _BINS // _CHUNK_BINS


def _coef_body(rois_ref, idx_ref, wgt_ref, *, size_h, size_w, n_hw):
    rois = rois_ref[...]
    r = lax.broadcasted_iota(jnp.int32, (1, _R), 1)
    bin_id = r // _RPB
    ph = (bin_id // _PW).astype(jnp.float32)
    pw = (bin_id % _PW).astype(jnp.float32)
    j = r % _RPB
    sy = (j // 8).astype(jnp.float32)
    sx = ((j // 4) % 2).astype(jnp.float32)
    cy = (j // 2) % 2
    cx = j % 2

    batch = rois[:, 0:1].astype(jnp.int32)
    start_w = rois[:, 1:2] * _SCALE
    start_h = rois[:, 2:3] * _SCALE
    end_w = rois[:, 3:4] * _SCALE
    end_h = rois[:, 4:5] * _SCALE
    roi_w = jnp.maximum(end_w - start_w, 1.0)
    roi_h = jnp.maximum(end_h - start_h, 1.0)
    bin_h = roi_h / _PH
    bin_w = roi_w / _PW
    y = start_h + (ph + (sy + 0.5) * 0.5) * bin_h
    x = start_w + (pw + (sx + 0.5) * 0.5) * bin_w

    def axis(coord, size, take_high):
        valid = (coord >= -1.0) & (coord <= float(size))
        c = jnp.maximum(coord, 0.0)
        low = jnp.minimum(jnp.floor(c).astype(jnp.int32), size - 1)
        high = jnp.minimum(low + 1, size - 1)
        frac = jnp.where(low >= size - 1, 0.0, c - low.astype(jnp.float32))
        sel = jnp.where(take_high == 1, high, low)
        wgt = jnp.where(take_high == 1, frac, 1.0 - frac)
        return sel, wgt, valid

    ysel, wy, vy = axis(y, size_h, cy)
    xsel, wx, vx = axis(x, size_w, cx)
    w = 0.25 * wy * wx * (vy & vx).astype(jnp.float32)
    idx_ref[...] = batch * n_hw + ysel * size_w + xsel
    wgt_ref[...] = w


def _coef(rois, size_h, size_w, n_hw):
    k = rois.shape[0]
    return pl.pallas_call(
        functools.partial(_coef_body, size_h=size_h, size_w=size_w, n_hw=n_hw),
        out_shape=[
            jax.ShapeDtypeStruct((k, _R), jnp.int32),
            jax.ShapeDtypeStruct((k, _R), jnp.float32),
        ],
    )(rois)


_XSEG = 112


def _make_sc_transpose(n, c, h, w):
    nseg = w // _XSEG
    units = n * h * nseg
    upt = units // _NW
    npairs = upt // 2
    mesh = plsc.VectorSubcoreMesh(
        core_axis_name="c", subcore_axis_name="s",
        num_cores=_NC, num_subcores=_NS)

    @functools.partial(
        pl.kernel,
        out_type=jax.ShapeDtypeStruct((n * h * w * c // 2,), jnp.int32),
        mesh=mesh,
        compiler_params=pltpu.CompilerParams(
            needs_layout_passes=False, use_tc_tiling_on_sc=False),
        scratch_types=[
            pltpu.VMEM((2, c, _XSEG), jnp.float32),
            pltpu.VMEM((2, _XSEG * c // 2), jnp.int32),
            pltpu.SemaphoreType.DMA,
            pltpu.SemaphoreType.DMA,
            pltpu.SemaphoreType.DMA,
            pltpu.SemaphoreType.DMA,
        ],
    )
    def tr_fn(in4, table, vin, vout, isem0, isem1, osem0, osem1):
        wid = lax.axis_index("s") * _NC + lax.axis_index("c")
        base = wid * upt
        isems = (isem0, isem1)
        osems = (osem0, osem1)

        def src_ref(u):
            uid = base + u
            nn = uid // (h * nseg)
            rem = lax.rem(uid, h * nseg)
            yy = rem // nseg
            xh = lax.rem(rem, nseg)
            return in4.at[nn, :, yy, pl.ds(xh * _XSEG, _XSEG)]

        def dst_ref(u):
            uid = base + u
            nn = uid // (h * nseg)
            rem = lax.rem(uid, h * nseg)
            yy = rem // nseg
            xh = lax.rem(rem, nseg)
            pix0 = (nn * h + yy) * w + xh * _XSEG
            return table.at[pl.ds(pix0 * (c // 2), _XSEG * (c // 2))]

        pltpu.async_copy(src_ref(0), vin.at[0], isems[0])
        pltpu.async_copy(src_ref(1), vin.at[1], isems[1])

        def pair_body(s, carry):
            for b in range(2):
                u = 2 * s + b
                pltpu.make_async_copy(src_ref(u), vin.at[b], isems[b]).wait()

                @pl.when(s >= 1)
                def _wait_prev_out(b=b, s=s):
                    pltpu.make_async_copy(
                        vout.at[b], dst_ref(2 * (s - 1) + b),
                        osems[b]).wait()

                def pg_body(pg, acc, b=b):
                    rowbase = (pg * _LANES
                               + lax.iota(jnp.int32, _LANES)) * (c // 2)
                    for ch in range(0, c, 2):
                        va = vin[b, ch, pl.ds(pg * _LANES, _LANES)]
                        vb = vin[b, ch + 1, pl.ds(pg * _LANES, _LANES)]
                        packed = plsc.pack(
                            va, vb, format=plsc.PackFormat.INTERLEAVED)
                        word = plsc.bitcast(packed, jnp.int32)
                        plsc.store_scatter(
                            vout.at[b], [rowbase + (ch // 2)], word)
                    return acc

                lax.fori_loop(0, _XSEG // _LANES, pg_body, 0, unroll=7)

                @pl.when(u + 2 < upt)
                def _issue_next(u=u, b=b):
                    pltpu.async_copy(src_ref(u + 2), vin.at[b], isems[b])

                pltpu.async_copy(vout.at[b], dst_ref(u), osems[b])
            return carry

        lax.fori_loop(0, npairs, pair_body, 0)
        for b in range(2):
            pltpu.make_async_copy(
                vout.at[b], dst_ref(upt - 2 + b), osems[b]).wait()

    return tr_fn


def _make_sc_kernel(k_rois, channels):
    rois_per_w = k_rois // _NW
    cpairs = channels // (2 * _LANES)
    mesh = plsc.VectorSubcoreMesh(
        core_axis_name="c", subcore_axis_name="s",
        num_cores=_NC, num_subcores=_NS)

    @functools.partial(
        pl.kernel,
        out_type=jax.ShapeDtypeStruct((k_rois, _BINS * channels), jnp.float32),
        mesh=mesh,
        compiler_params=pltpu.CompilerParams(
            needs_layout_passes=False, use_tc_tiling_on_sc=False),
        scratch_types=[
            pltpu.VMEM((2, _NCHUNKS, _CHUNK_ROWS), jnp.int32),
            pltpu.VMEM((2, _R), jnp.float32),
            pltpu.VMEM((3, _CHUNK_ROWS, channels // 2), jnp.int32),
            pltpu.VMEM((_BINS * channels,), jnp.float32),
            pltpu.SemaphoreType.DMA,
            pltpu.SemaphoreType.DMA,
            pltpu.SemaphoreType.DMA,
            pltpu.SemaphoreType.DMA,
            pltpu.SemaphoreType.DMA,
        ],
    )
    def sc_fn(table, idxs, wgts, out, idx_v, wgt_v, gbuf, obuf,
              isem, wsem, gsem0, gsem1, gsem2):
        wid = lax.axis_index("s") * _NC + lax.axis_index("c")
        k0 = wid * rois_per_w
        gsems = (gsem0, gsem1, gsem2)

        pltpu.async_copy(idxs.at[k0], idx_v.at[0], isem)
        pltpu.async_copy(wgts.at[k0], wgt_v.at[0], wsem)

        def roi_body(i, carry):
            k = k0 + i
            ib = lax.rem(i, 2)
            nib = 1 - ib
            pltpu.make_async_copy(idxs.at[k], idx_v.at[ib], isem).wait()
            pltpu.make_async_copy(wgts.at[k], wgt_v.at[ib], wsem).wait()

            @pl.when(i + 1 < rois_per_w)
            def _prefetch():
                pltpu.async_copy(idxs.at[k + 1], idx_v.at[nib], isem)
                pltpu.async_copy(wgts.at[k + 1], wgt_v.at[nib], wsem)

            descs = {}
            descs[0] = pltpu.async_copy(
                table.at[idx_v.at[ib, 0]], gbuf.at[0], gsems[0])
            descs[1] = pltpu.async_copy(
                table.at[idx_v.at[ib, 1]], gbuf.at[1], gsems[1])

            for c in range(_NCHUNKS):
                buf = c % 3
                if c + 2 < _NCHUNKS:
                    nb = (c + 2) % 3
                    descs[c + 2] = pltpu.async_copy(
                        table.at[idx_v.at[ib, c + 2]], gbuf.at[nb], gsems[nb])
                descs[c].wait()

                def bin_body(bl, acc_carry, c=c, buf=buf, ib=ib):
                    b = c * _CHUNK_BINS + bl
                    base_row = bl * _RPB
                    base_r = b * _RPB
                    acc_a = [jnp.zeros((_LANES,), jnp.float32)
                             for _ in range(cpairs)]
                    acc_b = [jnp.zeros((_LANES,), jnp.float32)
                             for _ in range(cpairs)]
                    for jj in range(_RPB):
                        wj = plsc.load_gather(
                            wgt_v.at[ib],
                            [jnp.broadcast_to(base_r + jj, (_LANES,))])
                        row = base_row + jj
                        for ci in range(cpairs):
                            word = gbuf[buf, row, pl.ds(ci * _LANES, _LANES)]
                            ea, eb = plsc.unpack(
                                plsc.bitcast(word, jnp.bfloat16),
                                format=plsc.PackFormat.INTERLEAVED,
                                preferred_element_type=jnp.float32)
                            acc_a[ci] = acc_a[ci] + wj * ea
                            acc_b[ci] = acc_b[ci] + wj * eb
                    ev2 = 2 * lax.iota(jnp.int32, _LANES)
                    for ci in range(cpairs):
                        cbase = b * channels + ci * 2 * _LANES
                        plsc.store_scatter(obuf, [cbase + ev2], acc_a[ci])
                        plsc.store_scatter(obuf, [cbase + ev2 + 1], acc_b[ci])
                    return acc_carry

                lax.fori_loop(0, _CHUNK_BINS, bin_body, 0)

            pltpu.sync_copy(obuf, out.at[k])
            return carry

        lax.fori_loop(0, rois_per_w, roi_body, 0)

    return sc_fn


@jax.jit
def kernel(input, rois):
    n, c, h, w = input.shape
    k = rois.shape[0]
    in_lin = jax.lax.optimization_barrier(input.reshape(-1))
    in4 = in_lin.reshape(n, c, h, w)
    table = _make_sc_transpose(n, c, h, w)(in4).reshape(n * h * w, c // 2)
    idx, wgt = _coef(rois, h, w, h * w)
    idx3 = jax.lax.optimization_barrier(
        idx.reshape(-1)).reshape(k, _NCHUNKS, _CHUNK_ROWS)
    wgt2 = jax.lax.optimization_barrier(wgt.reshape(-1)).reshape(k, _R)
    out = _make_sc_kernel(k, c)(table, idx3, wgt2).reshape(k, _BINS, c)
    return out.transpose(0, 2, 1).reshape(k, c, _PH, _PW)


# speedup vs baseline: 1.2105x; 1.0746x over previous
import functools

import jax
import jax.numpy as jnp
from jax import lax
from jax.experimental import pallas as pl
from jax.experimental.pallas import tpu as pltpu
from jax.experimental.pallas import tpu_sc as plsc

_PH = 7
_PW = 7
_SCALE = 224.0
_BINS = _PH * _PW
_RPB = 16
_R = _BINS * _RPB

_NC = 2
_NS = 16
_NW = _NC * _NS
_LANES = 16

_CHUNK_BINS = 7
_CHUNK_ROWS = _CHUNK_BINS * _RPB
_NCHUNKS = _BINS // _CHUNK_BINS


def _coef_body(rois_ref, idx_ref, wgt_ref, *, size_h, size_w, n_hw):
    rois = rois_ref[...]
    r = lax.broadcasted_iota(jnp.int32, (1, _R), 1)
    bin_id = r // _RPB
    ph = (bin_id // _PW).astype(jnp.float32)
    pw = (bin_id % _PW).astype(jnp.float32)
    j = r % _RPB
    sy = (j // 8).astype(jnp.float32)
    sx = ((j // 4) % 2).astype(jnp.float32)
    cy = (j // 2) % 2
    cx = j % 2

    batch = rois[:, 0:1].astype(jnp.int32)
    start_w = rois[:, 1:2] * _SCALE
    start_h = rois[:, 2:3] * _SCALE
    end_w = rois[:, 3:4] * _SCALE
    end_h = rois[:, 4:5] * _SCALE
    roi_w = jnp.maximum(end_w - start_w, 1.0)
    roi_h = jnp.maximum(end_h - start_h, 1.0)
    bin_h = roi_h / _PH
    bin_w = roi_w / _PW
    y = start_h + (ph + (sy + 0.5) * 0.5) * bin_h
    x = start_w + (pw + (sx + 0.5) * 0.5) * bin_w

    def axis(coord, size, take_high):
        valid = (coord >= -1.0) & (coord <= float(size))
        c = jnp.maximum(coord, 0.0)
        low = jnp.minimum(jnp.floor(c).astype(jnp.int32), size - 1)
        high = jnp.minimum(low + 1, size - 1)
        frac = jnp.where(low >= size - 1, 0.0, c - low.astype(jnp.float32))
        sel = jnp.where(take_high == 1, high, low)
        wgt = jnp.where(take_high == 1, frac, 1.0 - frac)
        return sel, wgt, valid

    ysel, wy, vy = axis(y, size_h, cy)
    xsel, wx, vx = axis(x, size_w, cx)
    w = 0.25 * wy * wx * (vy & vx).astype(jnp.float32)
    idx_ref[...] = batch * n_hw + ysel * size_w + xsel
    wgt_ref[...] = w


def _coef(rois, size_h, size_w, n_hw):
    k = rois.shape[0]
    return pl.pallas_call(
        functools.partial(_coef_body, size_h=size_h, size_w=size_w, n_hw=n_hw),
        out_shape=[
            jax.ShapeDtypeStruct((k, _R), jnp.int32),
            jax.ShapeDtypeStruct((k, _R), jnp.float32),
        ],
    )(rois)


_XSEG = 112


def _make_sc_transpose(n, c, h, w):
    nseg = w // _XSEG
    units = n * h * nseg
    upt = units // _NW
    npairs = upt // 2
    mesh = plsc.VectorSubcoreMesh(
        core_axis_name="c", subcore_axis_name="s",
        num_cores=_NC, num_subcores=_NS)

    @functools.partial(
        pl.kernel,
        out_type=jax.ShapeDtypeStruct((n * h * w * c // 2,), jnp.int32),
        mesh=mesh,
        compiler_params=pltpu.CompilerParams(
            needs_layout_passes=False, use_tc_tiling_on_sc=False),
        scratch_types=[
            pltpu.VMEM((2, c, _XSEG), jnp.float32),
            pltpu.VMEM((2, _XSEG * c // 2), jnp.int32),
            pltpu.SemaphoreType.DMA,
            pltpu.SemaphoreType.DMA,
            pltpu.SemaphoreType.DMA,
            pltpu.SemaphoreType.DMA,
        ],
    )
    def tr_fn(in4, table, vin, vout, isem0, isem1, osem0, osem1):
        wid = lax.axis_index("s") * _NC + lax.axis_index("c")
        base = wid * upt
        isems = (isem0, isem1)
        osems = (osem0, osem1)

        def src_ref(u):
            uid = base + u
            nn = uid // (h * nseg)
            rem = lax.rem(uid, h * nseg)
            yy = rem // nseg
            xh = lax.rem(rem, nseg)
            return in4.at[nn, :, yy, pl.ds(xh * _XSEG, _XSEG)]

        def dst_ref(u):
            uid = base + u
            nn = uid // (h * nseg)
            rem = lax.rem(uid, h * nseg)
            yy = rem // nseg
            xh = lax.rem(rem, nseg)
            pix0 = (nn * h + yy) * w + xh * _XSEG
            return table.at[pl.ds(pix0 * (c // 2), _XSEG * (c // 2))]

        pltpu.async_copy(src_ref(0), vin.at[0], isems[0])
        pltpu.async_copy(src_ref(1), vin.at[1], isems[1])

        def pair_body(s, carry):
            for b in range(2):
                u = 2 * s + b
                pltpu.make_async_copy(src_ref(u), vin.at[b], isems[b]).wait()

                @pl.when(s >= 1)
                def _wait_prev_out(b=b, s=s):
                    pltpu.make_async_copy(
                        vout.at[b], dst_ref(2 * (s - 1) + b),
                        osems[b]).wait()

                def pg_body(pg, acc, b=b):
                    rowbase = (pg * _LANES
                               + lax.iota(jnp.int32, _LANES)) * (c // 2)
                    for ch in range(0, c, 2):
                        va = vin[b, ch, pl.ds(pg * _LANES, _LANES)]
                        vb = vin[b, ch + 1, pl.ds(pg * _LANES, _LANES)]
                        packed = plsc.pack(
                            va, vb, format=plsc.PackFormat.INTERLEAVED)
                        word = plsc.bitcast(packed, jnp.int32)
                        plsc.store_scatter(
                            vout.at[b], [rowbase + (ch // 2)], word)
                    return acc

                lax.fori_loop(0, _XSEG // _LANES, pg_body, 0)

                @pl.when(u + 2 < upt)
                def _issue_next(u=u, b=b):
                    pltpu.async_copy(src_ref(u + 2), vin.at[b], isems[b])

                pltpu.async_copy(vout.at[b], dst_ref(u), osems[b])
            return carry

        lax.fori_loop(0, npairs, pair_body, 0)
        for b in range(2):
            pltpu.make_async_copy(
                vout.at[b], dst_ref(upt - 2 + b), osems[b]).wait()

    return tr_fn


def _make_sc_kernel(k_rois, channels):
    rois_per_w = k_rois // _NW
    cpairs = channels // (2 * _LANES)
    mesh = plsc.VectorSubcoreMesh(
        core_axis_name="c", subcore_axis_name="s",
        num_cores=_NC, num_subcores=_NS)

    @functools.partial(
        pl.kernel,
        out_type=jax.ShapeDtypeStruct((k_rois, _BINS * channels), jnp.float32),
        mesh=mesh,
        compiler_params=pltpu.CompilerParams(
            needs_layout_passes=False, use_tc_tiling_on_sc=False),
        scratch_types=[
            pltpu.VMEM((2, _NCHUNKS, _CHUNK_ROWS), jnp.int32),
            pltpu.VMEM((2, _R), jnp.float32),
            pltpu.VMEM((3, _CHUNK_ROWS, channels // 2), jnp.int32),
            pltpu.VMEM((_BINS * channels,), jnp.float32),
            pltpu.SemaphoreType.DMA,
            pltpu.SemaphoreType.DMA,
            pltpu.SemaphoreType.DMA,
            pltpu.SemaphoreType.DMA,
            pltpu.SemaphoreType.DMA,
        ],
    )
    def sc_fn(table, idxs, wgts, out, idx_v, wgt_v, gbuf, obuf,
              isem, wsem, gsem0, gsem1, gsem2):
        wid = lax.axis_index("s") * _NC + lax.axis_index("c")
        k0 = wid * rois_per_w
        gsems = (gsem0, gsem1, gsem2)

        pltpu.async_copy(idxs.at[k0], idx_v.at[0], isem)
        pltpu.async_copy(wgts.at[k0], wgt_v.at[0], wsem)

        def roi_body(i, carry):
            k = k0 + i
            ib = lax.rem(i, 2)
            nib = 1 - ib
            pltpu.make_async_copy(idxs.at[k], idx_v.at[ib], isem).wait()
            pltpu.make_async_copy(wgts.at[k], wgt_v.at[ib], wsem).wait()

            @pl.when(i + 1 < rois_per_w)
            def _prefetch():
                pltpu.async_copy(idxs.at[k + 1], idx_v.at[nib], isem)
                pltpu.async_copy(wgts.at[k + 1], wgt_v.at[nib], wsem)

            descs = {}
            descs[0] = pltpu.async_copy(
                table.at[idx_v.at[ib, 0]], gbuf.at[0], gsems[0])
            descs[1] = pltpu.async_copy(
                table.at[idx_v.at[ib, 1]], gbuf.at[1], gsems[1])

            for c in range(_NCHUNKS):
                buf = c % 3
                if c + 2 < _NCHUNKS:
                    nb = (c + 2) % 3
                    descs[c + 2] = pltpu.async_copy(
                        table.at[idx_v.at[ib, c + 2]], gbuf.at[nb], gsems[nb])
                descs[c].wait()

                def bin_body(bl, acc_carry, c=c, buf=buf, ib=ib):
                    b = c * _CHUNK_BINS + bl
                    base_row = bl * _RPB
                    base_r = b * _RPB
                    acc_a = [jnp.zeros((_LANES,), jnp.float32)
                             for _ in range(cpairs)]
                    acc_b = [jnp.zeros((_LANES,), jnp.float32)
                             for _ in range(cpairs)]
                    for jj in range(_RPB):
                        wj = plsc.load_gather(
                            wgt_v.at[ib],
                            [jnp.broadcast_to(base_r + jj, (_LANES,))])
                        row = base_row + jj
                        for ci in range(cpairs):
                            word = gbuf[buf, row, pl.ds(ci * _LANES, _LANES)]
                            ea, eb = plsc.unpack(
                                plsc.bitcast(word, jnp.bfloat16),
                                format=plsc.PackFormat.INTERLEAVED,
                                preferred_element_type=jnp.float32)
                            acc_a[ci] = acc_a[ci] + wj * ea
                            acc_b[ci] = acc_b[ci] + wj * eb
                    ev2 = 2 * lax.iota(jnp.int32, _LANES)
                    for ci in range(cpairs):
                        cbase = b * channels + ci * 2 * _LANES
                        plsc.store_scatter(obuf, [cbase + ev2], acc_a[ci])
                        plsc.store_scatter(obuf, [cbase + ev2 + 1], acc_b[ci])
                    return acc_carry

                lax.fori_loop(0, _CHUNK_BINS, bin_body, 0)

            pltpu.sync_copy(obuf, out.at[k])
            return carry

        lax.fori_loop(0, rois_per_w, roi_body, 0)

    return sc_fn


@jax.jit
def kernel(input, rois):
    n, c, h, w = input.shape
    k = rois.shape[0]
    in_lin = jax.lax.optimization_barrier(input.reshape(-1))
    in4 = in_lin.reshape(n, c, h, w)
    table = _make_sc_transpose(n, c, h, w)(in4).reshape(n * h * w, c // 2)
    idx, wgt = _coef(rois, h, w, h * w)
    idx3 = jax.lax.optimization_barrier(
        idx.reshape(-1)).reshape(k, _NCHUNKS, _CHUNK_ROWS)
    wgt2 = jax.lax.optimization_barrier(wgt.reshape(-1)).reshape(k, _R)
    out = _make_sc_kernel(k, c)(table, idx3, wgt2).reshape(k, _BINS, c)
    return out.transpose(0, 2, 1).reshape(k, c, _PH, _PW)


# speedup vs baseline: 1.2270x; 1.0136x over previous
import functools

import jax
import jax.numpy as jnp
from jax import lax
from jax.experimental import pallas as pl
from jax.experimental.pallas import tpu as pltpu
from jax.experimental.pallas import tpu_sc as plsc

_PH = 7
_PW = 7
_SCALE = 224.0
_BINS = _PH * _PW
_RPB = 16
_R = _BINS * _RPB

_NC = 2
_NS = 16
_NW = _NC * _NS
_LANES = 16

_CHUNK_BINS = 7
_CHUNK_ROWS = _CHUNK_BINS * _RPB
_NCHUNKS = _BINS // _CHUNK_BINS


def _coef_body(rois_ref, idx_ref, wgt_ref, *, size_h, size_w, n_hw):
    rois = rois_ref[...]
    r = lax.broadcasted_iota(jnp.int32, (1, _R), 1)
    bin_id = r // _RPB
    ph = (bin_id // _PW).astype(jnp.float32)
    pw = (bin_id % _PW).astype(jnp.float32)
    j = r % _RPB
    sy = (j // 8).astype(jnp.float32)
    sx = ((j // 4) % 2).astype(jnp.float32)
    cy = (j // 2) % 2
    cx = j % 2

    batch = rois[:, 0:1].astype(jnp.int32)
    start_w = rois[:, 1:2] * _SCALE
    start_h = rois[:, 2:3] * _SCALE
    end_w = rois[:, 3:4] * _SCALE
    end_h = rois[:, 4:5] * _SCALE
    roi_w = jnp.maximum(end_w - start_w, 1.0)
    roi_h = jnp.maximum(end_h - start_h, 1.0)
    bin_h = roi_h / _PH
    bin_w = roi_w / _PW
    y = start_h + (ph + (sy + 0.5) * 0.5) * bin_h
    x = start_w + (pw + (sx + 0.5) * 0.5) * bin_w

    def axis(coord, size, take_high):
        valid = (coord >= -1.0) & (coord <= float(size))
        c = jnp.maximum(coord, 0.0)
        low = jnp.minimum(jnp.floor(c).astype(jnp.int32), size - 1)
        high = jnp.minimum(low + 1, size - 1)
        frac = jnp.where(low >= size - 1, 0.0, c - low.astype(jnp.float32))
        sel = jnp.where(take_high == 1, high, low)
        wgt = jnp.where(take_high == 1, frac, 1.0 - frac)
        return sel, wgt, valid

    ysel, wy, vy = axis(y, size_h, cy)
    xsel, wx, vx = axis(x, size_w, cx)
    w = 0.25 * wy * wx * (vy & vx).astype(jnp.float32)
    idx_ref[...] = batch * n_hw + ysel * size_w + xsel
    wgt_ref[...] = w


def _coef(rois, size_h, size_w, n_hw):
    k = rois.shape[0]
    return pl.pallas_call(
        functools.partial(_coef_body, size_h=size_h, size_w=size_w, n_hw=n_hw),
        out_shape=[
            jax.ShapeDtypeStruct((k, _R), jnp.int32),
            jax.ShapeDtypeStruct((k, _R), jnp.float32),
        ],
    )(rois)


_XSEG = 112


def _make_sc_transpose(n, c, h, w):
    nseg = w // _XSEG
    units = n * h * nseg
    upt = units // _NW
    npairs = upt // 2
    mesh = plsc.VectorSubcoreMesh(
        core_axis_name="c", subcore_axis_name="s",
        num_cores=_NC, num_subcores=_NS)

    @functools.partial(
        pl.kernel,
        out_type=jax.ShapeDtypeStruct((n * h * w * c // 2,), jnp.int32),
        mesh=mesh,
        compiler_params=pltpu.CompilerParams(
            needs_layout_passes=False, use_tc_tiling_on_sc=False),
        scratch_types=[
            pltpu.VMEM((2, c, _XSEG), jnp.float32),
            pltpu.VMEM((2, _XSEG * c // 2), jnp.int32),
            pltpu.SemaphoreType.DMA,
            pltpu.SemaphoreType.DMA,
            pltpu.SemaphoreType.DMA,
            pltpu.SemaphoreType.DMA,
        ],
    )
    def tr_fn(in4, table, vin, vout, isem0, isem1, osem0, osem1):
        wid = lax.axis_index("s") * _NC + lax.axis_index("c")
        base = wid * upt
        isems = (isem0, isem1)
        osems = (osem0, osem1)

        def src_ref(u):
            uid = base + u
            nn = uid // (h * nseg)
            rem = lax.rem(uid, h * nseg)
            yy = rem // nseg
            xh = lax.rem(rem, nseg)
            return in4.at[nn, :, yy, pl.ds(xh * _XSEG, _XSEG)]

        def dst_ref(u):
            uid = base + u
            nn = uid // (h * nseg)
            rem = lax.rem(uid, h * nseg)
            yy = rem // nseg
            xh = lax.rem(rem, nseg)
            pix0 = (nn * h + yy) * w + xh * _XSEG
            return table.at[pl.ds(pix0 * (c // 2), _XSEG * (c // 2))]

        pltpu.async_copy(src_ref(0), vin.at[0], isems[0])
        pltpu.async_copy(src_ref(1), vin.at[1], isems[1])

        def pair_body(s, carry):
            for b in range(2):
                u = 2 * s + b
                pltpu.make_async_copy(src_ref(u), vin.at[b], isems[b]).wait()

                @pl.when(s >= 1)
                def _wait_prev_out(b=b, s=s):
                    pltpu.make_async_copy(
                        vout.at[b], dst_ref(2 * (s - 1) + b),
                        osems[b]).wait()

                def pg_body(pg, acc, b=b):
                    rowbase = (pg * _LANES
                               + lax.iota(jnp.int32, _LANES)) * (c // 2)
                    for ch in range(0, c, 2):
                        va = vin[b, ch, pl.ds(pg * _LANES, _LANES)]
                        vb = vin[b, ch + 1, pl.ds(pg * _LANES, _LANES)]
                        packed = plsc.pack(
                            va, vb, format=plsc.PackFormat.INTERLEAVED)
                        word = plsc.bitcast(packed, jnp.int32)
                        plsc.store_scatter(
                            vout.at[b], [rowbase + (ch // 2)], word)
                    return acc

                lax.fori_loop(0, _XSEG // _LANES, pg_body, 0)

                @pl.when(u + 2 < upt)
                def _issue_next(u=u, b=b):
                    pltpu.async_copy(src_ref(u + 2), vin.at[b], isems[b])

                pltpu.async_copy(vout.at[b], dst_ref(u), osems[b])
            return carry

        lax.fori_loop(0, npairs, pair_body, 0)
        for b in range(2):
            pltpu.make_async_copy(
                vout.at[b], dst_ref(upt - 2 + b), osems[b]).wait()

    return tr_fn


def _make_sc_kernel(k_rois, channels):
    rois_per_w = k_rois // _NW
    cpairs = channels // (2 * _LANES)
    mesh = plsc.VectorSubcoreMesh(
        core_axis_name="c", subcore_axis_name="s",
        num_cores=_NC, num_subcores=_NS)

    @functools.partial(
        pl.kernel,
        out_type=jax.ShapeDtypeStruct((k_rois, _BINS * channels), jnp.float32),
        mesh=mesh,
        compiler_params=pltpu.CompilerParams(
            needs_layout_passes=False, use_tc_tiling_on_sc=False),
        scratch_types=[
            pltpu.VMEM((2, _NCHUNKS, _CHUNK_ROWS), jnp.int32),
            pltpu.VMEM((2, _R), jnp.float32),
            pltpu.VMEM((4, _CHUNK_ROWS, channels // 2), jnp.int32),
            pltpu.VMEM((_BINS * channels,), jnp.float32),
            pltpu.SemaphoreType.DMA,
            pltpu.SemaphoreType.DMA,
            pltpu.SemaphoreType.DMA,
            pltpu.SemaphoreType.DMA,
            pltpu.SemaphoreType.DMA,
            pltpu.SemaphoreType.DMA,
        ],
    )
    def sc_fn(table, idxs, wgts, out, idx_v, wgt_v, gbuf, obuf,
              isem, wsem, gsem0, gsem1, gsem2, gsem3):
        wid = lax.axis_index("s") * _NC + lax.axis_index("c")
        k0 = wid * rois_per_w
        gsems = (gsem0, gsem1, gsem2, gsem3)

        pltpu.async_copy(idxs.at[k0], idx_v.at[0], isem)
        pltpu.async_copy(wgts.at[k0], wgt_v.at[0], wsem)

        def roi_body(i, carry):
            k = k0 + i
            ib = lax.rem(i, 2)
            nib = 1 - ib
            pltpu.make_async_copy(idxs.at[k], idx_v.at[ib], isem).wait()
            pltpu.make_async_copy(wgts.at[k], wgt_v.at[ib], wsem).wait()

            @pl.when(i + 1 < rois_per_w)
            def _prefetch():
                pltpu.async_copy(idxs.at[k + 1], idx_v.at[nib], isem)
                pltpu.async_copy(wgts.at[k + 1], wgt_v.at[nib], wsem)

            descs = {}
            for c0 in range(3):
                descs[c0] = pltpu.async_copy(
                    table.at[idx_v.at[ib, c0]], gbuf.at[c0], gsems[c0])

            for c in range(_NCHUNKS):
                buf = c % 4
                if c + 3 < _NCHUNKS:
                    nb = (c + 3) % 4
                    descs[c + 3] = pltpu.async_copy(
                        table.at[idx_v.at[ib, c + 3]], gbuf.at[nb], gsems[nb])
                descs[c].wait()

                def bin_body(bl, acc_carry, c=c, buf=buf, ib=ib):
                    b = c * _CHUNK_BINS + bl
                    base_row = bl * _RPB
                    base_r = b * _RPB
                    acc_a = [jnp.zeros((_LANES,), jnp.float32)
                             for _ in range(cpairs)]
                    acc_b = [jnp.zeros((_LANES,), jnp.float32)
                             for _ in range(cpairs)]
                    for jj in range(_RPB):
                        wj = plsc.load_gather(
                            wgt_v.at[ib],
                            [jnp.broadcast_to(base_r + jj, (_LANES,))])
                        row = base_row + jj
                        for ci in range(cpairs):
                            word = gbuf[buf, row, pl.ds(ci * _LANES, _LANES)]
                            ea, eb = plsc.unpack(
                                plsc.bitcast(word, jnp.bfloat16),
                                format=plsc.PackFormat.INTERLEAVED,
                                preferred_element_type=jnp.float32)
                            acc_a[ci] = acc_a[ci] + wj * ea
                            acc_b[ci] = acc_b[ci] + wj * eb
                    ev2 = 2 * lax.iota(jnp.int32, _LANES)
                    for ci in range(cpairs):
                        cbase = b * channels + ci * 2 * _LANES
                        plsc.store_scatter(obuf, [cbase + ev2], acc_a[ci])
                        plsc.store_scatter(obuf, [cbase + ev2 + 1], acc_b[ci])
                    return acc_carry

                lax.fori_loop(0, _CHUNK_BINS, bin_body, 0)

            pltpu.sync_copy(obuf, out.at[k])
            return carry

        lax.fori_loop(0, rois_per_w, roi_body, 0)

    return sc_fn


@jax.jit
def kernel(input, rois):
    n, c, h, w = input.shape
    k = rois.shape[0]
    in_lin = jax.lax.optimization_barrier(input.reshape(-1))
    in4 = in_lin.reshape(n, c, h, w)
    table = _make_sc_transpose(n, c, h, w)(in4).reshape(n * h * w, c // 2)
    idx, wgt = _coef(rois, h, w, h * w)
    idx3 = jax.lax.optimization_barrier(
        idx.reshape(-1)).reshape(k, _NCHUNKS, _CHUNK_ROWS)
    wgt2 = jax.lax.optimization_barrier(wgt.reshape(-1)).reshape(k, _R)
    out = _make_sc_kernel(k, c)(table, idx3, wgt2).reshape(k, _BINS, c)
    return out.transpose(0, 2, 1).reshape(k, c, _PH, _PW)
